# speedup vs baseline: 1.0635x; 1.0635x over previous
import functools

import jax
import jax.numpy as jnp
from jax import lax
from jax.experimental import pallas as pl

HIDDEN = 4096
N_EXPERTS = 64
TOP_K = 8
ROW_BLOCK = 512


def _router_body(x_ref, wt_ref, idx_ref, w_ref, p_ref):
    x_blk = x_ref[...]
    w_t = wt_ref[...]
    logits = jnp.dot(x_blk, w_t, preferred_element_type=jnp.float32)

    m = jnp.max(logits, axis=1, keepdims=True)
    e = jnp.exp(logits - m)
    probs = e / jnp.sum(e, axis=1, keepdims=True)
    p_ref[...] = probs

    iota = lax.broadcasted_iota(jnp.int32, probs.shape, 1)
    vals = probs
    wt_cols = []
    idx_cols = []
    for _ in range(TOP_K):
        mx = jnp.max(vals, axis=1, keepdims=True)
        cand = jnp.where(vals == mx, iota, N_EXPERTS)
        amin = jnp.min(cand, axis=1, keepdims=True)
        wt_cols.append(mx)
        idx_cols.append(amin)
        vals = jnp.where(iota == amin, -jnp.inf, vals)

    weights = jnp.concatenate(wt_cols, axis=1)
    weights = weights / (jnp.sum(weights, axis=1, keepdims=True) + 1e-9)
    idx_ref[...] = jnp.concatenate(idx_cols, axis=1)
    w_ref[...] = weights


@jax.jit
def _router(flat, w_t):
    n_rows = flat.shape[0]
    grid = (n_rows // ROW_BLOCK,)
    return pl.pallas_call(
        _router_body,
        grid=grid,
        in_specs=[
            pl.BlockSpec((ROW_BLOCK, HIDDEN), lambda i: (i, 0)),
            pl.BlockSpec((HIDDEN, N_EXPERTS), lambda i: (0, 0)),
        ],
        out_specs=[
            pl.BlockSpec((ROW_BLOCK, TOP_K), lambda i: (i, 0)),
            pl.BlockSpec((ROW_BLOCK, TOP_K), lambda i: (i, 0)),
            pl.BlockSpec((ROW_BLOCK, N_EXPERTS), lambda i: (i, 0)),
        ],
        out_shape=[
            jax.ShapeDtypeStruct((n_rows, TOP_K), jnp.int32),
            jax.ShapeDtypeStruct((n_rows, TOP_K), jnp.float32),
            jax.ShapeDtypeStruct((n_rows, N_EXPERTS), jnp.float32),
        ],
    )(flat, w_t)


def kernel(x, W):
    flat = x.reshape(-1, x.shape[-1])
    indices, weights, probs = _router(flat, W.T)
    return indices, weights.astype(x.dtype), probs
